# baseline (speedup 1.0000x reference)
import functools

import jax
import jax.numpy as jnp
from jax import lax
from jax.experimental import pallas as pl
from jax.experimental.pallas import tpu as pltpu
from jax.experimental.pallas import tpu_sc as plsc

EMB_SIZE = 100000
EMB_DIM = 64
BATCH = 16384

NUM_CORES = 2
NUM_SUBCORES = 16
NUM_WORKERS = NUM_CORES * NUM_SUBCORES
B_PER_W = BATCH // NUM_WORKERS
IDX_CHUNK = 128
N_CHUNKS = B_PER_W // IDX_CHUNK


def _sc_gather_body(p1_hbm, p2_hbm, emb_hbm, bias_hbm,
                    rows1_hbm, rows2_hbm, b1_hbm, b2_hbm,
                    idx1_v, idx2_v, rows1_v, rows2_v, bv1_v, bv2_v, sem):
    wid = lax.axis_index("s") * NUM_CORES + lax.axis_index("c")
    base = wid * B_PER_W

    pltpu.sync_copy(p1_hbm.at[pl.ds(base, B_PER_W)], idx1_v)
    pltpu.sync_copy(p2_hbm.at[pl.ds(base, B_PER_W)], idx2_v)

    copies = []
    for j in range(N_CHUNKS):
        sl = pl.ds(j * IDX_CHUNK, IDX_CHUNK)
        copies.append(pltpu.async_copy(
            emb_hbm.at[idx1_v.at[sl]], rows1_v.at[sl], sem))
        copies.append(pltpu.async_copy(
            emb_hbm.at[idx2_v.at[sl]], rows2_v.at[sl], sem))
        copies.append(pltpu.async_copy(
            bias_hbm.at[idx1_v.at[sl]], bv1_v.at[sl], sem))
        copies.append(pltpu.async_copy(
            bias_hbm.at[idx2_v.at[sl]], bv2_v.at[sl], sem))
    for c in copies:
        c.wait()

    pltpu.sync_copy(rows1_v, rows1_hbm.at[pl.ds(base, B_PER_W)])
    pltpu.sync_copy(rows2_v, rows2_hbm.at[pl.ds(base, B_PER_W)])
    pltpu.sync_copy(bv1_v, b1_hbm.at[pl.ds(base, B_PER_W)])
    pltpu.sync_copy(bv2_v, b2_hbm.at[pl.ds(base, B_PER_W)])


_sc_gather = pl.kernel(
    _sc_gather_body,
    out_type=(
        jax.ShapeDtypeStruct((BATCH, EMB_DIM), jnp.float32),
        jax.ShapeDtypeStruct((BATCH, EMB_DIM), jnp.float32),
        jax.ShapeDtypeStruct((BATCH,), jnp.float32),
        jax.ShapeDtypeStruct((BATCH,), jnp.float32),
    ),
    mesh=plsc.VectorSubcoreMesh(core_axis_name="c", subcore_axis_name="s"),
    scratch_types=[
        pltpu.VMEM((B_PER_W,), jnp.int32),
        pltpu.VMEM((B_PER_W,), jnp.int32),
        pltpu.VMEM((B_PER_W, EMB_DIM), jnp.float32),
        pltpu.VMEM((B_PER_W, EMB_DIM), jnp.float32),
        pltpu.VMEM((B_PER_W,), jnp.float32),
        pltpu.VMEM((B_PER_W,), jnp.float32),
        pltpu.SemaphoreType.DMA,
    ],
    compiler_params=pltpu.CompilerParams(use_tc_tiling_on_sc=False),
)


def _tc_body(bias_ref, r1_ref, r2_ref, b1_ref, b2_ref, out_ref):
    inter = jnp.sum(r1_ref[...] * r2_ref[...], axis=1)
    z = inter + b1_ref[...] + b2_ref[...] + bias_ref[0]
    out_ref[...] = jax.nn.sigmoid(z)


TC_BLK = 2048


def _tc_compute(rows1, rows2, b1, b2, bias):
    grid = BATCH // TC_BLK
    return pl.pallas_call(
        _tc_body,
        grid=(grid,),
        in_specs=[
            pl.BlockSpec(memory_space=pltpu.SMEM),
            pl.BlockSpec((TC_BLK, EMB_DIM), lambda i: (i, 0)),
            pl.BlockSpec((TC_BLK, EMB_DIM), lambda i: (i, 0)),
            pl.BlockSpec((TC_BLK,), lambda i: (i,)),
            pl.BlockSpec((TC_BLK,), lambda i: (i,)),
        ],
        out_specs=pl.BlockSpec((TC_BLK,), lambda i: (i,)),
        out_shape=jax.ShapeDtypeStruct((BATCH,), jnp.float32),
    )(bias, rows1, rows2, b1, b2)


@jax.jit
def kernel(product1, product2, product_embedding, product_bias, bias):
    p1 = product1.astype(jnp.int32)
    p2 = product2.astype(jnp.int32)
    bias_flat = product_bias.reshape(EMB_SIZE)
    rows1, rows2, b1, b2 = _sc_gather(p1, p2, product_embedding, bias_flat)
    return _tc_compute(rows1, rows2, b1, b2, bias)


# speedup vs baseline: 1.2033x; 1.2033x over previous
import jax
import jax.numpy as jnp
import numpy as np
from jax import lax
from jax.experimental import pallas as pl
from jax.experimental.pallas import tpu as pltpu
from jax.experimental.pallas import tpu_sc as plsc

EMB_SIZE = 100000
EMB_DIM = 64
BATCH = 16384

NUM_CORES = 2
NUM_SUBCORES = 16
NUM_WORKERS = NUM_CORES * NUM_SUBCORES
B_PER_W = BATCH // NUM_WORKERS
IDX_CHUNK = 128
N_CHUNKS = B_PER_W // IDX_CHUNK
GROUPS_PER_CHUNK = IDX_CHUNK // 16

def _dot16(rows1_f, rows2_f, base_elem, lane):
    row_idx = base_elem + lane
    d_vec = lane
    acc = jnp.zeros((16,), jnp.float32)
    for t in range(EMB_DIM):
        a = plsc.load_gather(rows1_f, [row_idx, d_vec])
        b = plsc.load_gather(rows2_f, [row_idx, d_vec])
        acc = acc + a * b
        d_vec = (d_vec + 1) % EMB_DIM
    return acc


def _sc_body(p1_hbm, p2_hbm, emb_hbm, bias_hbm, b0_hbm, out_hbm,
             idx1_v, idx2_v, rows1_f, rows2_f, bv1_v, bv2_v, b0_v, out_v,
             sem_b, *sems):
    wid = lax.axis_index("s") * NUM_CORES + lax.axis_index("c")
    base = wid * B_PER_W
    lane = lax.iota(jnp.int32, 16)

    pltpu.sync_copy(p1_hbm.at[pl.ds(base, B_PER_W)], idx1_v)
    pltpu.sync_copy(p2_hbm.at[pl.ds(base, B_PER_W)], idx2_v)
    pltpu.sync_copy(b0_hbm, b0_v)

    copies = []
    for c in range(N_CHUNKS):
        sl = pl.ds(c * IDX_CHUNK, IDX_CHUNK)
        copies.append((
            pltpu.async_copy(emb_hbm.at[idx1_v.at[sl]], rows1_f.at[sl],
                             sems[c]),
            pltpu.async_copy(emb_hbm.at[idx2_v.at[sl]], rows2_f.at[sl],
                             sems[c]),
        ))
    cb1 = pltpu.async_copy(bias_hbm.at[idx1_v], bv1_v, sem_b)
    cb2 = pltpu.async_copy(bias_hbm.at[idx2_v], bv2_v, sem_b)
    cb1.wait()
    cb2.wait()
    b0 = b0_v[...]

    for c in range(N_CHUNKS):
        copies[c][0].wait()
        copies[c][1].wait()

        def group_body(g, _):
            e0 = c * IDX_CHUNK + g * 16
            inter = _dot16(rows1_f, rows2_f, e0, lane)
            b1 = bv1_v[pl.ds(e0, 16)]
            b2 = bv2_v[pl.ds(e0, 16)]
            z = inter + b1 + b2 + b0
            out_v[pl.ds(e0, 16)] = 1.0 / (1.0 + jnp.exp(-z))
            return 0

        lax.fori_loop(0, GROUPS_PER_CHUNK, group_body, 0)

    pltpu.sync_copy(out_v, out_hbm.at[pl.ds(base, B_PER_W)])


_sc_fused = pl.kernel(
    _sc_body,
    out_type=jax.ShapeDtypeStruct((BATCH,), jnp.float32),
    mesh=plsc.VectorSubcoreMesh(core_axis_name="c", subcore_axis_name="s"),
    scratch_types=[
        pltpu.VMEM((B_PER_W,), jnp.int32),
        pltpu.VMEM((B_PER_W,), jnp.int32),
        pltpu.VMEM((B_PER_W, EMB_DIM), jnp.float32),
        pltpu.VMEM((B_PER_W, EMB_DIM), jnp.float32),
        pltpu.VMEM((B_PER_W,), jnp.float32),
        pltpu.VMEM((B_PER_W,), jnp.float32),
        pltpu.VMEM((16,), jnp.float32),
        pltpu.VMEM((B_PER_W,), jnp.float32),
        pltpu.SemaphoreType.DMA,
    ] + [pltpu.SemaphoreType.DMA] * N_CHUNKS,
    compiler_params=pltpu.CompilerParams(use_tc_tiling_on_sc=False,
                                         needs_layout_passes=False),
)


@jax.jit
def kernel(product1, product2, product_embedding, product_bias, bias):
    p1 = product1.astype(jnp.int32)
    p2 = product2.astype(jnp.int32)
    bias_flat = product_bias.reshape(EMB_SIZE)
    bias16 = jnp.broadcast_to(bias, (16,))
    return _sc_fused(p1, p2, product_embedding, bias_flat, bias16)


# speedup vs baseline: 1.2118x; 1.0070x over previous
import jax
import jax.numpy as jnp
import numpy as np
from jax import lax
from jax.experimental import pallas as pl
from jax.experimental.pallas import tpu as pltpu
from jax.experimental.pallas import tpu_sc as plsc

EMB_SIZE = 100000
EMB_DIM = 64
BATCH = 16384

NUM_CORES = 2
NUM_SUBCORES = 16
NUM_WORKERS = NUM_CORES * NUM_SUBCORES
B_PER_W = BATCH // NUM_WORKERS
IDX_CHUNK = 64
N_CHUNKS = B_PER_W // IDX_CHUNK
GROUPS_PER_CHUNK = IDX_CHUNK // 16

def _dot16(rows1_f, rows2_f, base_elem, lane):
    row_idx = base_elem + lane
    d_vec = lane
    acc = jnp.zeros((16,), jnp.float32)
    for t in range(EMB_DIM):
        a = plsc.load_gather(rows1_f, [row_idx, d_vec])
        b = plsc.load_gather(rows2_f, [row_idx, d_vec])
        acc = acc + a * b
        d_vec = (d_vec + 1) % EMB_DIM
    return acc


def _sc_body(p1_hbm, p2_hbm, emb_hbm, bias_hbm, b0_hbm, out_hbm,
             idx1_v, idx2_v, rows1_f, rows2_f, bv1_v, bv2_v, b0_v, out_v,
             sem_b, *sems):
    wid = lax.axis_index("s") * NUM_CORES + lax.axis_index("c")
    base = wid * B_PER_W
    lane = lax.iota(jnp.int32, 16)

    pltpu.sync_copy(p1_hbm.at[pl.ds(base, B_PER_W)], idx1_v)
    pltpu.sync_copy(p2_hbm.at[pl.ds(base, B_PER_W)], idx2_v)
    pltpu.sync_copy(b0_hbm, b0_v)

    copies = []
    for c in range(N_CHUNKS):
        sl = pl.ds(c * IDX_CHUNK, IDX_CHUNK)
        copies.append((
            pltpu.async_copy(emb_hbm.at[idx1_v.at[sl]], rows1_f.at[sl],
                             sems[c]),
            pltpu.async_copy(emb_hbm.at[idx2_v.at[sl]], rows2_f.at[sl],
                             sems[c]),
        ))
    cb1 = pltpu.async_copy(bias_hbm.at[idx1_v], bv1_v, sem_b)
    cb2 = pltpu.async_copy(bias_hbm.at[idx2_v], bv2_v, sem_b)

    for c in range(N_CHUNKS):
        copies[c][0].wait()
        copies[c][1].wait()

        def group_body(g, _):
            e0 = c * IDX_CHUNK + g * 16
            out_v[pl.ds(e0, 16)] = _dot16(rows1_f, rows2_f, e0, lane)
            return 0

        lax.fori_loop(0, GROUPS_PER_CHUNK, group_body, 0)

    cb1.wait()
    cb2.wait()
    b0 = b0_v[...]

    def bias_body(g, _):
        e0 = g * 16
        z = out_v[pl.ds(e0, 16)] + bv1_v[pl.ds(e0, 16)] \
            + bv2_v[pl.ds(e0, 16)] + b0
        out_v[pl.ds(e0, 16)] = 1.0 / (1.0 + jnp.exp(-z))
        return 0

    lax.fori_loop(0, B_PER_W // 16, bias_body, 0)

    pltpu.sync_copy(out_v, out_hbm.at[pl.ds(base, B_PER_W)])


_sc_fused = pl.kernel(
    _sc_body,
    out_type=jax.ShapeDtypeStruct((BATCH,), jnp.float32),
    mesh=plsc.VectorSubcoreMesh(core_axis_name="c", subcore_axis_name="s"),
    scratch_types=[
        pltpu.VMEM((B_PER_W,), jnp.int32),
        pltpu.VMEM((B_PER_W,), jnp.int32),
        pltpu.VMEM((B_PER_W, EMB_DIM), jnp.float32),
        pltpu.VMEM((B_PER_W, EMB_DIM), jnp.float32),
        pltpu.VMEM((B_PER_W,), jnp.float32),
        pltpu.VMEM((B_PER_W,), jnp.float32),
        pltpu.VMEM((16,), jnp.float32),
        pltpu.VMEM((B_PER_W,), jnp.float32),
        pltpu.SemaphoreType.DMA,
    ] + [pltpu.SemaphoreType.DMA] * N_CHUNKS,
    compiler_params=pltpu.CompilerParams(use_tc_tiling_on_sc=False,
                                         needs_layout_passes=False),
)


@jax.jit
def kernel(product1, product2, product_embedding, product_bias, bias):
    p1 = product1.astype(jnp.int32)
    p2 = product2.astype(jnp.int32)
    bias_flat = product_bias.reshape(EMB_SIZE)
    bias16 = jnp.broadcast_to(bias, (16,))
    return _sc_fused(p1, p2, product_embedding, bias_flat, bias16)
